# Initial kernel scaffold; baseline (speedup 1.0000x reference)
import jax
import jax.numpy as jnp
from jax.experimental import pallas as pl


def kernel(atomic_num, bond_length, edge_index, graph_ids, W_atom, W_bond, W_gin, b_gin, eps, W_mlp, b_mlp):
    raise NotImplementedError("write your pallas kernel here")


# speedup vs baseline: 2.5561x; 2.5561x over previous
import functools

import jax
import jax.numpy as jnp
from jax import lax
from jax.experimental import pallas as pl
from jax.experimental.pallas import tpu as pltpu
from jax.experimental.pallas import tpu_sc as plsc

N = 10000
E = 320000
G = 64
D = 128
DEPTH = 3

NC = 2
NS = 16
NW = NC * NS
EPW = E // NW
CHUNK = 80
NCHUNK = EPW // CHUNK
RPT = 624
REM = N - NS * RPT
ZR = 208


def _edge_body(h_hbm, e_hbm, src_hbm, dst_hbm, out_hbm,
               src_v, dst_v, e_v, h_v, z_v, sem, agg_sh):
    cid = lax.axis_index("c")
    sid = lax.axis_index("s")
    wid = cid * NS + sid

    def zrow(i, _):
        for r in range(D // 16):
            z_v[i, pl.ds(r * 16, 16)] = jnp.zeros((16,), jnp.float32)
        return 0
    lax.fori_loop(0, ZR, zrow, 0)
    for k in range(RPT // ZR):
        pltpu.sync_copy(z_v, agg_sh.at[pl.ds(sid * RPT + k * ZR, ZR)])

    @pl.when(sid == NS - 1)
    def _zero_rem():
        pltpu.sync_copy(z_v.at[pl.ds(0, REM)], agg_sh.at[pl.ds(NS * RPT, REM)])

    plsc.subcore_barrier()

    def chunk(k, _):
        base = wid * EPW + k * CHUNK
        pltpu.sync_copy(src_hbm.at[pl.ds(base, CHUNK)], src_v)
        pltpu.sync_copy(dst_hbm.at[pl.ds(base, CHUNK)], dst_v)
        pltpu.sync_copy(e_hbm.at[pl.ds(base, CHUNK)], e_v)
        pltpu.async_copy(h_hbm.at[src_v], h_v, sem).wait()

        def edge(j, _):
            for r in range(D // 16):
                s = pl.ds(r * 16, 16)
                h_v[j, s] = jnp.maximum(h_v[j, s] + e_v[j, s], 0.0)
            return 0
        lax.fori_loop(0, CHUNK, edge, 0)
        pltpu.sync_copy(h_v, agg_sh.at[dst_v], add=True)
        return 0
    lax.fori_loop(0, NCHUNK, chunk, 0)

    plsc.subcore_barrier()
    pltpu.sync_copy(agg_sh.at[pl.ds(sid * RPT, RPT)],
                    out_hbm.at[cid, pl.ds(sid * RPT, RPT)])

    @pl.when(sid == NS - 1)
    def _write_rem():
        pltpu.sync_copy(agg_sh.at[pl.ds(NS * RPT, REM)],
                        out_hbm.at[cid, pl.ds(NS * RPT, REM)])


@functools.cache
def _edge_kernel():
    return pl.kernel(
        _edge_body,
        out_type=jax.ShapeDtypeStruct((NC, N, D), jnp.float32),
        mesh=plsc.VectorSubcoreMesh(core_axis_name="c", subcore_axis_name="s"),
        scratch_types=[
            pltpu.VMEM((CHUNK,), jnp.int32),
            pltpu.VMEM((CHUNK,), jnp.int32),
            pltpu.VMEM((CHUNK, D), jnp.float32),
            pltpu.VMEM((CHUNK, D), jnp.float32),
            pltpu.VMEM((ZR, D), jnp.float32),
            pltpu.SemaphoreType.DMA,
            pltpu.VMEM_SHARED((N, D), jnp.float32),
        ],
    )


def _mm_body(x_ref, w_ref, o_ref):
    o_ref[...] = jnp.dot(x_ref[...], w_ref[...],
                         preferred_element_type=jnp.float32)


def _matmul(x, w, blk):
    m, k = x.shape
    d = w.shape[1]
    return pl.pallas_call(
        _mm_body,
        grid=(m // blk,),
        in_specs=[
            pl.BlockSpec((blk, k), lambda i: (i, 0)),
            pl.BlockSpec((k, d), lambda i: (0, 0)),
        ],
        out_specs=pl.BlockSpec((blk, d), lambda i: (i, 0)),
        out_shape=jax.ShapeDtypeStruct((m, d), jnp.float32),
    )(x, w)


def _upd_body(s_ref, h_ref, a0_ref, a1_ref, ad_ref, w_ref, b_ref, o_ref):
    rst = s_ref[0] * h_ref[...] + (a0_ref[...] + a1_ref[...])
    hn = jnp.dot(rst, w_ref[...], preferred_element_type=jnp.float32)
    o_ref[...] = jnp.maximum(hn + b_ref[...], 0.0) + ad_ref[...]


def _update(scale, h, a0, a1, ad, w, b, blk=1000):
    return pl.pallas_call(
        _upd_body,
        grid=(N // blk,),
        in_specs=[
            pl.BlockSpec(memory_space=pltpu.SMEM),
            pl.BlockSpec((blk, D), lambda i: (i, 0)),
            pl.BlockSpec((blk, D), lambda i: (i, 0)),
            pl.BlockSpec((blk, D), lambda i: (i, 0)),
            pl.BlockSpec((blk, D), lambda i: (i, 0)),
            pl.BlockSpec((D, D), lambda i: (0, 0)),
            pl.BlockSpec((1, D), lambda i: (0, 0)),
        ],
        out_specs=pl.BlockSpec((blk, D), lambda i: (i, 0)),
        out_shape=jax.ShapeDtypeStruct((N, D), jnp.float32),
    )(scale, h, a0, a1, ad, w, b)


def _pool_body(ad_ref, gid_ref, wm_ref, bm_ref, o_ref):
    ids = gid_ref[...]
    iota = lax.broadcasted_iota(jnp.int32, (G, N), 0)
    onehot = jnp.where(iota == ids, 1.0, 0.0)
    sums = jnp.dot(onehot, ad_ref[...], preferred_element_type=jnp.float32)
    cnts = jnp.sum(onehot, axis=1, keepdims=True)
    feat = sums / jnp.maximum(cnts, 1.0)
    o_ref[...] = jnp.dot(feat, wm_ref[...],
                         preferred_element_type=jnp.float32) + bm_ref[...]


def _pool(ad, gids, wm, bm):
    return pl.pallas_call(
        _pool_body,
        out_shape=jax.ShapeDtypeStruct((G, 1), jnp.float32),
    )(ad, gids.reshape(1, N), wm, bm.reshape(1, 1))


def kernel(atomic_num, bond_length, edge_index, graph_ids,
           W_atom, W_bond, W_gin, b_gin, eps, W_mlp, b_mlp):
    src = edge_index[0]
    dst = edge_index[1]

    h = _matmul(atomic_num, W_atom, blk=1000)
    e = _matmul(bond_length, W_bond, blk=4000)

    atom_dense = h
    for i in range(DEPTH):
        agg2 = _edge_kernel()(h, e, src, dst)
        scale = (1.0 + eps[i]).reshape(1)
        h = _update(scale, h, agg2[0], agg2[1], atom_dense,
                    W_gin[i], b_gin[i].reshape(1, D))
        atom_dense = h

    out = _pool(atom_dense, graph_ids, W_mlp, b_mlp)
    return out.reshape(G)


# speedup vs baseline: 3.6992x; 1.4472x over previous
import functools

import jax
import jax.numpy as jnp
from jax import lax
from jax.experimental import pallas as pl
from jax.experimental.pallas import tpu as pltpu
from jax.experimental.pallas import tpu_sc as plsc

N = 10000
E = 320000
G = 64
D = 128
DEPTH = 3

NC = 2
NS = 16
NW = NC * NS
EPW = E // NW
CHUNK = 40
NCHUNK = EPW // CHUNK
RPT = 624
REM = N - NS * RPT
ZR = 104


NBUF = 3


def _edge_body(h_hbm, e_hbm, src_hbm, dst_hbm, out_hbm,
               src_v, dst_v, e_v, h_v, z_v, semA, semB, semS, agg_sh):
    cid = lax.axis_index("c")
    sid = lax.axis_index("s")
    wid = cid * NS + sid
    ebase = wid * EPW

    def zrow(i, _):
        for r in range(D // 16):
            z_v[i, pl.ds(r * 16, 16)] = jnp.zeros((16,), jnp.float32)
        return 0
    lax.fori_loop(0, ZR, zrow, 0)
    for k in range(RPT // ZR):
        pltpu.sync_copy(z_v, agg_sh.at[pl.ds(sid * RPT + k * ZR, ZR)])

    @pl.when(sid == NS - 1)
    def _zero_rem():
        pltpu.sync_copy(z_v.at[pl.ds(0, REM)], agg_sh.at[pl.ds(NS * RPT, REM)])

    plsc.subcore_barrier()

    def stage(g, b):
        base = ebase + g * CHUNK
        pltpu.sync_copy(src_hbm.at[pl.ds(base, CHUNK)], src_v.at[b])
        pltpu.sync_copy(dst_hbm.at[pl.ds(base, CHUNK)], dst_v.at[b])
        pltpu.async_copy(e_hbm.at[pl.ds(base, CHUNK)], e_v.at[b], semA.at[b])
        pltpu.async_copy(h_hbm.at[src_v.at[b]], h_v.at[b], semB.at[b])

    stage(0, 0)
    stage(1, 1)

    def chunk(g, _):
        b0 = lax.rem(g, NBUF)
        b2 = lax.rem(g + 2, NBUF)
        base0 = ebase + g * CHUNK
        pltpu.make_async_copy(e_hbm.at[pl.ds(base0, CHUNK)],
                              e_v.at[b0], semA.at[b0]).wait()
        pltpu.make_async_copy(h_hbm.at[src_v.at[b0]],
                              h_v.at[b0], semB.at[b0]).wait()

        @plsc.parallel_loop(0, CHUNK, unroll=2)
        def _compute(j):
            for r in range(D // 16):
                s = pl.ds(r * 16, 16)
                h_v[b0, j, s] = jnp.maximum(h_v[b0, j, s] + e_v[b0, j, s], 0.0)

        pltpu.async_copy(h_v.at[b0], agg_sh.at[dst_v.at[b0]],
                         semS.at[b0], add=True)

        @pl.when(g + 2 < NCHUNK)
        def _pref():
            @pl.when(g >= 1)
            def _ws():
                pltpu.make_async_copy(h_v.at[b2], agg_sh.at[dst_v.at[b2]],
                                      semS.at[b2]).wait()
            stage(g + 2, b2)
        return 0
    lax.fori_loop(0, NCHUNK, chunk, 0)

    for k in range(NBUF):
        b = (NCHUNK - NBUF + k) % NBUF
        pltpu.make_async_copy(h_v.at[b], agg_sh.at[dst_v.at[b]],
                              semS.at[b]).wait()

    plsc.subcore_barrier()
    pltpu.sync_copy(agg_sh.at[pl.ds(sid * RPT, RPT)],
                    out_hbm.at[cid, pl.ds(sid * RPT, RPT)])

    @pl.when(sid == NS - 1)
    def _write_rem():
        pltpu.sync_copy(agg_sh.at[pl.ds(NS * RPT, REM)],
                        out_hbm.at[cid, pl.ds(NS * RPT, REM)])


@functools.cache
def _edge_kernel():
    return pl.kernel(
        _edge_body,
        out_type=jax.ShapeDtypeStruct((NC, N, D), jnp.float32),
        mesh=plsc.VectorSubcoreMesh(core_axis_name="c", subcore_axis_name="s"),
        scratch_types=[
            pltpu.VMEM((NBUF, CHUNK), jnp.int32),
            pltpu.VMEM((NBUF, CHUNK), jnp.int32),
            pltpu.VMEM((NBUF, CHUNK, D), jnp.float32),
            pltpu.VMEM((NBUF, CHUNK, D), jnp.float32),
            pltpu.VMEM((ZR, D), jnp.float32),
            pltpu.SemaphoreType.DMA((NBUF,)),
            pltpu.SemaphoreType.DMA((NBUF,)),
            pltpu.SemaphoreType.DMA((NBUF,)),
            pltpu.VMEM_SHARED((N, D), jnp.float32),
        ],
    )


def _mm_body(x_ref, w_ref, o_ref):
    o_ref[...] = jnp.dot(x_ref[...], w_ref[...],
                         preferred_element_type=jnp.float32)


def _matmul(x, w, blk):
    m, k = x.shape
    d = w.shape[1]
    return pl.pallas_call(
        _mm_body,
        grid=(m // blk,),
        in_specs=[
            pl.BlockSpec((blk, k), lambda i: (i, 0)),
            pl.BlockSpec((k, d), lambda i: (0, 0)),
        ],
        out_specs=pl.BlockSpec((blk, d), lambda i: (i, 0)),
        out_shape=jax.ShapeDtypeStruct((m, d), jnp.float32),
    )(x, w)


def _upd_body(s_ref, h_ref, a0_ref, a1_ref, ad_ref, w_ref, b_ref, o_ref):
    rst = s_ref[0] * h_ref[...] + (a0_ref[...] + a1_ref[...])
    hn = jnp.dot(rst, w_ref[...], preferred_element_type=jnp.float32)
    o_ref[...] = jnp.maximum(hn + b_ref[...], 0.0) + ad_ref[...]


def _update(scale, h, a0, a1, ad, w, b, blk=1000):
    return pl.pallas_call(
        _upd_body,
        grid=(N // blk,),
        in_specs=[
            pl.BlockSpec(memory_space=pltpu.SMEM),
            pl.BlockSpec((blk, D), lambda i: (i, 0)),
            pl.BlockSpec((blk, D), lambda i: (i, 0)),
            pl.BlockSpec((blk, D), lambda i: (i, 0)),
            pl.BlockSpec((blk, D), lambda i: (i, 0)),
            pl.BlockSpec((D, D), lambda i: (0, 0)),
            pl.BlockSpec((1, D), lambda i: (0, 0)),
        ],
        out_specs=pl.BlockSpec((blk, D), lambda i: (i, 0)),
        out_shape=jax.ShapeDtypeStruct((N, D), jnp.float32),
    )(scale, h, a0, a1, ad, w, b)


def _pool_body(ad_ref, gid_ref, wm_ref, bm_ref, o_ref):
    ids = gid_ref[...]
    iota = lax.broadcasted_iota(jnp.int32, (G, N), 0)
    onehot = jnp.where(iota == ids, 1.0, 0.0)
    sums = jnp.dot(onehot, ad_ref[...], preferred_element_type=jnp.float32)
    cnts = jnp.sum(onehot, axis=1, keepdims=True)
    feat = sums / jnp.maximum(cnts, 1.0)
    o_ref[...] = jnp.dot(feat, wm_ref[...],
                         preferred_element_type=jnp.float32) + bm_ref[...]


def _pool(ad, gids, wm, bm):
    return pl.pallas_call(
        _pool_body,
        out_shape=jax.ShapeDtypeStruct((G, 1), jnp.float32),
    )(ad, gids.reshape(1, N), wm, bm.reshape(1, 1))


def kernel(atomic_num, bond_length, edge_index, graph_ids,
           W_atom, W_bond, W_gin, b_gin, eps, W_mlp, b_mlp):
    src = edge_index[0]
    dst = edge_index[1]

    h = _matmul(atomic_num, W_atom, blk=1000)
    e = _matmul(bond_length, W_bond, blk=4000)

    atom_dense = h
    for i in range(DEPTH):
        agg2 = _edge_kernel()(h, e, src, dst)
        scale = (1.0 + eps[i]).reshape(1)
        h = _update(scale, h, agg2[0], agg2[1], atom_dense,
                    W_gin[i], b_gin[i].reshape(1, D))
        atom_dense = h

    out = _pool(atom_dense, graph_ids, W_mlp, b_mlp)
    return out.reshape(G)
